# baseline (speedup 1.0000x reference)
import functools
import math

import jax
import jax.numpy as jnp
from jax import lax
from jax.experimental import pallas as pl
from jax.experimental.pallas import tpu as pltpu
from jax.experimental.pallas import tpu_sc as plsc

_NC = 2
_NS = 16
_NW = _NC * _NS
_LANES = 16
_BBLK = 128


def _make_emb_kernel(B, L, V, D):
    n_bblk = B // _BBLK
    n_lgrp = _NW // n_bblk
    l_per_w = L // n_lgrp
    scale = math.sqrt(D)
    nk = _BBLK // _LANES
    dpk = D // nk
    mesh = plsc.VectorSubcoreMesh(core_axis_name="c", subcore_axis_name="s")

    scratch = (
        [pltpu.VMEM((l_per_w, _BBLK), jnp.int32)]
        + [pltpu.VMEM((_BBLK, D), jnp.float32) for _ in range(2)]
        + [pltpu.VMEM((D, _BBLK), jnp.float32) for _ in range(2)]
        + [pltpu.SemaphoreType.DMA for _ in range(4)]
    )

    @functools.partial(
        pl.kernel,
        out_type=jax.ShapeDtypeStruct((L, D, B), jnp.float32),
        mesh=mesh,
        scratch_types=scratch,
        compiler_params=pltpu.CompilerParams(needs_layout_passes=False),
    )
    def emb(table_hbm, ids_hbm, out_hbm, idsv, rb0, rb1, tb0, tb1,
            g0, g1, o0, o1):
        rbuf = (rb0, rb1)
        tbuf = (tb0, tb1)
        sem_g = (g0, g1)
        sem_o = (o0, o1)
        wid = lax.axis_index("s") * _NC + lax.axis_index("c")
        lgrp = wid // n_bblk
        b0 = (wid % n_bblk) * _BBLK
        l0 = lgrp * l_per_w

        pltpu.sync_copy(ids_hbm.at[lgrp, :, pl.ds(b0, _BBLK)], idsv)

        rowi = [lax.iota(jnp.int32, _LANES) + g * _LANES for g in range(nk)]

        def enqueue16(u, buf, k):
            vec = idsv[u, pl.ds(k * _LANES, _LANES)]
            for lane in range(_LANES):
                r = vec[lane]
                pltpu.async_copy(
                    table_hbm.at[pl.ds(r, 1)],
                    rbuf[buf].at[pl.ds(k * _LANES + lane, 1)],
                    sem_g[buf],
                )

        def transpose8(src, dst, k):
            for dd in range(dpk):
                d = k * dpk + dd
                coli = jnp.full((_LANES,), d, jnp.int32)
                for g in range(nk):
                    v = plsc.load_gather(src, [rowi[g], coli])
                    dst[d, pl.ds(g * _LANES, _LANES)] = v * scale

        def fire_unit(u, buf):
            @plsc.parallel_loop(0, nk, unroll=2)
            def _(k):
                enqueue16(u, buf, k)

        def merged(u, buf):
            src = rbuf[1 - buf]
            dst = tbuf[1 - buf]

            @plsc.parallel_loop(0, nk)
            def _(k):
                enqueue16(u, buf, k)
                transpose8(src, dst, k)

        def transpose_only(buf):
            src = rbuf[buf]
            dst = tbuf[buf]

            @plsc.parallel_loop(0, nk, unroll=2)
            def _(k):
                transpose8(src, dst, k)

        def drain_unit(buf):
            pltpu.make_async_copy(
                table_hbm.at[pl.ds(0, _BBLK)], rbuf[buf], sem_g[buf]).wait()

        def fire_out(u, buf):
            pltpu.async_copy(
                tbuf[buf], out_hbm.at[l0 + u, :, pl.ds(b0, _BBLK)],
                sem_o[buf])

        def wait_out(u, buf):
            pltpu.make_async_copy(
                tbuf[buf], out_hbm.at[l0 + u, :, pl.ds(b0, _BBLK)],
                sem_o[buf]).wait()

        def step(u, buf, first=False):
            if not first:
                wait_out(u - 2, buf)
            drain_unit(1 - buf)
            merged(u, buf)
            fire_out(u - 1, 1 - buf)

        fire_unit(0, 0)
        step(1, 1, first=True)

        def pair_body(t, carry):
            for buf in (0, 1):
                u = 2 * t + (2 if buf == 0 else 3)
                step(u, buf)
            return carry

        lax.fori_loop(0, (l_per_w - 2) // 2, pair_body, 0)

        u = l_per_w
        buf = u % 2
        wait_out(u - 2, buf)
        drain_unit(1 - buf)
        transpose_only(1 - buf)
        fire_out(u - 1, 1 - buf)
        wait_out(u - 1, 1 - buf)

    return emb


def kernel(input, table):
    B, L = input.shape
    V, D = table.shape
    n_lgrp = _NW // (B // _BBLK)
    ids3 = input.T.reshape(n_lgrp, L // n_lgrp, B).astype(jnp.int32)
    emb = _make_emb_kernel(B, L, V, D)
    out = emb(table, ids3)
    return jnp.transpose(out, (2, 0, 1))


# speedup vs baseline: 1.0274x; 1.0274x over previous
import functools
import math

import jax
import jax.numpy as jnp
from jax import lax
from jax.experimental import pallas as pl
from jax.experimental.pallas import tpu as pltpu
from jax.experimental.pallas import tpu_sc as plsc

_NC = 2
_NS = 16
_NW = _NC * _NS
_LANES = 16
_BBLK = 128


def _make_emb_kernel(B, L, V, D):
    n_bblk = B // _BBLK
    n_lgrp = _NW // n_bblk
    l_per_w = L // n_lgrp
    scale = math.sqrt(D)
    mesh = plsc.VectorSubcoreMesh(core_axis_name="c", subcore_axis_name="s")

    scratch = (
        [pltpu.VMEM((l_per_w, _BBLK), jnp.int32)]
        + [pltpu.VMEM((_BBLK, D), jnp.float32) for _ in range(2)]
        + [pltpu.VMEM((D, _BBLK), jnp.float32) for _ in range(2)]
        + [pltpu.SemaphoreType.DMA for _ in range(6)]
    )

    @functools.partial(
        pl.kernel,
        out_type=jax.ShapeDtypeStruct((L, D, B), jnp.float32),
        mesh=mesh,
        scratch_types=scratch,
        compiler_params=pltpu.CompilerParams(needs_layout_passes=False),
    )
    def emb(table_hbm, ids_hbm, out_hbm, idsv, rb0, rb1, tb0, tb1,
            g0, g1, g2, g3, o0, o1):
        rbuf = (rb0, rb1)
        tbuf = (tb0, tb1)
        sem_g = ((g0, g2), (g1, g3))
        sem_o = (o0, o1)
        wid = lax.axis_index("s") * _NC + lax.axis_index("c")
        lgrp = wid // n_bblk
        b0 = (wid % n_bblk) * _BBLK
        l0 = lgrp * l_per_w

        pltpu.sync_copy(ids_hbm.at[lgrp, :, pl.ds(b0, _BBLK)], idsv)

        def fire_unit(u, buf):
            @plsc.parallel_loop(0, _BBLK // _LANES, unroll=4)
            def fire16(k):
                vec = idsv[u, pl.ds(k * _LANES, _LANES)]
                for lane in range(_LANES):
                    r = vec[lane]
                    pltpu.async_copy(
                        table_hbm.at[pl.ds(r, 1)],
                        rbuf[buf].at[pl.ds(k * _LANES + lane, 1)],
                        sem_g[buf][lane % 2],
                    )

        def drain_unit(buf):
            half = rbuf[buf].at[pl.ds(0, _BBLK // 2)]
            pltpu.make_async_copy(
                table_hbm.at[pl.ds(0, _BBLK // 2)], half,
                sem_g[buf][0]).wait()
            pltpu.make_async_copy(
                table_hbm.at[pl.ds(0, _BBLK // 2)], half,
                sem_g[buf][1]).wait()

        def transpose_scale(buf):
            src = rbuf[buf]
            dst = tbuf[buf]
            rowi = [lax.iota(jnp.int32, _LANES) + g * _LANES
                    for g in range(_BBLK // _LANES)]

            @plsc.parallel_loop(0, D, unroll=8)
            def body_d(d):
                coli = jnp.full((_LANES,), d, jnp.int32)
                for g in range(_BBLK // _LANES):
                    v = plsc.load_gather(src, [rowi[g], coli])
                    dst[d, pl.ds(g * _LANES, _LANES)] = v * scale

        def fire_out(u, buf):
            pltpu.async_copy(
                tbuf[buf], out_hbm.at[l0 + u, :, pl.ds(b0, _BBLK)],
                sem_o[buf])

        def wait_out(u, buf):
            pltpu.make_async_copy(
                tbuf[buf], out_hbm.at[l0 + u, :, pl.ds(b0, _BBLK)],
                sem_o[buf]).wait()

        fire_unit(0, 0)

        def pair_body(t, carry):
            for buf in (1, 0):
                u = 2 * t + (1 if buf == 1 else 2)

                @pl.when(u >= 2)
                def _():
                    wait_out(u - 2, buf)

                @pl.when(u < l_per_w)
                def _():
                    fire_unit(u, buf)
                drain_unit(1 - buf)
                transpose_scale(1 - buf)
                fire_out(u - 1, 1 - buf)
            return carry

        lax.fori_loop(0, l_per_w // 2, pair_body, 0)
        wait_out(l_per_w - 1, (l_per_w - 1) % 2)

    return emb


def kernel(input, table):
    B, L = input.shape
    V, D = table.shape
    n_lgrp = _NW // (B // _BBLK)
    ids3 = input.T.reshape(n_lgrp, L // n_lgrp, B).astype(jnp.int32)
    emb = _make_emb_kernel(B, L, V, D)
    out = emb(table, ids3)
    return jnp.transpose(out, (2, 0, 1))
